# Initial kernel scaffold; baseline (speedup 1.0000x reference)
import jax
import jax.numpy as jnp
from jax.experimental import pallas as pl


def kernel(input_tensor, W, b):
    raise NotImplementedError("write your pallas kernel here")


# speedup vs baseline: 18.4740x; 18.4740x over previous
import functools

import jax
import jax.numpy as jnp
from jax.experimental import pallas as pl
from jax.experimental.pallas import tpu as pltpu

_K = 16


def _encoder_kernel(x_ref, wx_ref, woff_ref, wh_ref, bias_ref, out_ref,
                    h_scr, c_scr, *, T, B, C, HD, N):
    f32 = jnp.float32
    bf16 = jnp.bfloat16
    BIG = f32(3.0e38)
    NEG = f32(-3.0e38)

    h_scr[...] = jnp.zeros((B, HD, N), f32)
    c_scr[...] = jnp.zeros((B, HD, N), f32)
    wx = wx_ref[...]
    woff = woff_ref[...]
    wh = wh_ref[...]
    bias = bias_ref[...]

    def step(t, carry):
        tp = jnp.maximum(t - 1, 0)
        A = []
        bm_hi, bm_lo, ch_hi, ch_lo = [], [], [], []
        for bi in range(B):
            xt = x_ref[t, bi]
            pos_t = xt[:4]
            pos_prev = x_ref[tp, bi, :4]
            a = (jnp.dot(wx, xt, preferred_element_type=f32)
                 - jnp.dot(woff, pos_t, preferred_element_type=f32) + bias)
            bm = (jnp.dot(woff, pos_prev, preferred_element_type=f32)
                  + jnp.dot(wh, h_scr[bi], preferred_element_type=f32))
            A.append(a)
            hi = bm.astype(bf16)
            bm_hi.append(hi)
            bm_lo.append((bm - hi.astype(f32)).astype(bf16))
            cprev = c_scr[bi]
            chi = cprev.astype(bf16)
            ch_hi.append(chi)
            ch_lo.append((cprev - chi.astype(f32)).astype(bf16))

        pos_q = x_ref[t, :, :4]
        pos_r = x_ref[tp, :, :4]
        dist = jnp.zeros((B, N, N), f32)
        for d in range(4):
            diff = pos_r[:, d, :, None] - pos_q[:, d, None, :]
            dist = dist + diff * diff

        iota = jax.lax.broadcasted_iota(jnp.int32, (B, N, N), 1)

        def kstep(_, kcarry):
            dist_c, h_acc, c_acc = kcarry
            val = jnp.min(dist_c, axis=1, keepdims=True)
            cand = dist_c == val
            midx = jnp.min(jnp.where(cand, iota, N), axis=1, keepdims=True)
            onehot = iota == midx
            dist_c = jnp.where(onehot, BIG, dist_c)
            oh = onehot.astype(bf16)
            h_new, c_new = [], []
            for bi in range(B):
                ohb = oh[bi]
                g = (jnp.dot(bm_hi[bi], ohb, preferred_element_type=f32)
                     + jnp.dot(bm_lo[bi], ohb, preferred_element_type=f32)
                     + A[bi])
                cnb = (jnp.dot(ch_hi[bi], ohb, preferred_element_type=f32)
                       + jnp.dot(ch_lo[bi], ohb, preferred_element_type=f32))
                ig = jax.nn.sigmoid(g[0 * HD:1 * HD])
                fg = jax.nn.sigmoid(g[1 * HD:2 * HD])
                og = jax.nn.sigmoid(g[2 * HD:3 * HD])
                gg = jnp.tanh(g[3 * HD:4 * HD])
                cn = fg * cnb + ig * gg
                hn = og * jnp.tanh(cn)
                h_new.append(jnp.maximum(h_acc[bi], hn))
                c_new.append(jnp.maximum(c_acc[bi], cn))
            return dist_c, tuple(h_new), tuple(c_new)

        init = (dist,
                tuple(jnp.full((HD, N), NEG, f32) for _ in range(B)),
                tuple(jnp.full((HD, N), NEG, f32) for _ in range(B)))
        _, h_fin, c_fin = jax.lax.fori_loop(0, _K, kstep, init)
        for bi in range(B):
            h_scr[bi] = h_fin[bi]
            c_scr[bi] = c_fin[bi]
            out_ref[t, bi] = h_fin[bi]
        return carry

    jax.lax.fori_loop(0, T, step, 0)


@jax.jit
def kernel(input_tensor, W, b):
    B, T, C, N = input_tensor.shape
    O = W.shape[0]
    HD = O // 4
    x_t = jnp.transpose(input_tensor, (1, 0, 2, 3))
    wx = W[:, :C]
    woff = W[:, C:C + 4]
    wh = W[:, C + 4:]
    bias2 = jnp.broadcast_to(b[:, None], (O, N))
    kern = functools.partial(_encoder_kernel, T=T, B=B, C=C, HD=HD, N=N)
    houts = pl.pallas_call(
        kern,
        out_shape=jax.ShapeDtypeStruct((T, B, HD, N), jnp.float32),
        scratch_shapes=[
            pltpu.VMEM((B, HD, N), jnp.float32),
            pltpu.VMEM((B, HD, N), jnp.float32),
        ],
    )(x_t, wx, woff, wh, bias2)
    pos = input_tensor[:, :, :4]
    return jnp.concatenate([pos, jnp.transpose(houts, (1, 0, 2, 3))], axis=2)
